# Initial kernel scaffold; baseline (speedup 1.0000x reference)
import jax
import jax.numpy as jnp
from jax.experimental import pallas as pl


def kernel(boxes, scores):
    raise NotImplementedError("write your pallas kernel here")


# speedup vs baseline: 16.6405x; 16.6405x over previous
import jax
import jax.numpy as jnp
from jax.experimental import pallas as pl

N = 20000
NPAD = 160 * 128
ROWS = 160
LANES = 128
PRE_NMS_TOPK = 1000
MAX_DET = 300
IOU_THRESH = 0.5
SCORE_THRESH = 0.05


def _nms_kernel(x1_ref, y1_ref, x2_ref, y2_ref, s_ref, out_ref):
    x1 = x1_ref[...]
    y1 = y1_ref[...]
    x2 = x2_ref[...]
    y2 = y2_ref[...]
    s = s_ref[...]
    sb = jax.lax.bitcast_convert_type(s, jnp.int32)

    def bs_step(_, carry):
        lo, hi = carry
        mid = lo + ((hi - lo) >> 1)
        cnt = jnp.sum((sb >= mid).astype(jnp.int32))
        ge = cnt >= PRE_NMS_TOPK
        lo = jnp.where(ge, mid, lo)
        hi = jnp.where(ge, hi, mid)
        return lo, hi

    lo0 = jnp.int32(0)
    hi0 = jnp.int32(0x7F800000)
    lo, _ = jax.lax.fori_loop(0, 31, bs_step, (lo0, hi0))

    valid0 = (sb >= lo) & (s > SCORE_THRESH)
    sm0 = jnp.where(valid0, s, -1.0)

    areas = (x2 - x1) * (y2 - y1)
    ii = jax.lax.broadcasted_iota(jnp.int32, (ROWS, LANES), 0)
    jj = jax.lax.broadcasted_iota(jnp.int32, (ROWS, LANES), 1)
    flat = ii * LANES + jj
    lane = jax.lax.broadcasted_iota(jnp.int32, (1, LANES), 1)

    def nms_step(k, sm):
        m = jnp.max(sm)
        any_valid = m > 0.0
        eq = sm == m
        idx = jnp.min(jnp.where(eq & any_valid, flat, jnp.int32(NPAD)))
        sel = flat == idx
        bx1 = jnp.sum(jnp.where(sel, x1, 0.0))
        by1 = jnp.sum(jnp.where(sel, y1, 0.0))
        bx2 = jnp.sum(jnp.where(sel, x2, 0.0))
        by2 = jnp.sum(jnp.where(sel, y2, 0.0))
        bs_ = jnp.where(any_valid, m, 0.0)

        xx1 = jnp.maximum(bx1, x1)
        yy1 = jnp.maximum(by1, y1)
        xx2 = jnp.minimum(bx2, x2)
        yy2 = jnp.minimum(by2, y2)
        inter = jnp.maximum(xx2 - xx1, 0.0) * jnp.maximum(yy2 - yy1, 0.0)
        barea = (bx2 - bx1) * (by2 - by1)
        iou = inter / (barea + areas - inter + 1e-9)
        sm = jnp.where(iou < IOU_THRESH, sm, -1.0)

        row = jnp.where(lane == 0, bx1, 0.0)
        row = jnp.where(lane == 1, by1, row)
        row = jnp.where(lane == 2, bx2, row)
        row = jnp.where(lane == 3, by2, row)
        row = jnp.where(lane == 4, bs_, row)
        out_ref[pl.ds(k, 1), :] = row
        return sm

    jax.lax.fori_loop(0, MAX_DET, nms_step, sm0)


def kernel(boxes, scores):
    pad = NPAD - N
    s = jnp.concatenate([scores, jnp.full((pad,), -1.0, jnp.float32)])
    b = jnp.concatenate([boxes, jnp.zeros((pad, 4), jnp.float32)], axis=0)
    s = s.reshape(ROWS, LANES)
    x1 = b[:, 0].reshape(ROWS, LANES)
    y1 = b[:, 1].reshape(ROWS, LANES)
    x2 = b[:, 2].reshape(ROWS, LANES)
    y2 = b[:, 3].reshape(ROWS, LANES)

    out = pl.pallas_call(
        _nms_kernel,
        out_shape=jax.ShapeDtypeStruct((304, LANES), jnp.float32),
    )(x1, y1, x2, y2, s)
    return out[:MAX_DET, :5]
